# baseline (speedup 1.0000x reference)
import jax
import jax.numpy as jnp
from jax.experimental import pallas as pl
from jax.experimental.pallas import tpu as pltpu
from jax.experimental.pallas import tpu_sc as plsc

NUM_K = 8192
DIM = 64
BATCH = 8
TOKENS_PER_ROW = 1024
N_TOKENS = BATCH * TOKENS_PER_ROW
KBLK = 2048
TBLK = 2048


def _block_argmin(dist, bmin, k):
    iota = jax.lax.broadcasted_iota(
        jnp.int32, dist.shape, 1).astype(jnp.float32)
    barg_f = jnp.min(jnp.where(dist == bmin, iota, jnp.float32(2.0 ** 30)),
                     axis=1, keepdims=True)
    return barg_f.astype(jnp.int32) + k * KBLK


def _fast_body(u_ref, wt2_ref, a_ref, min_ref, arg_ref):
    t = pl.program_id(0)
    k = pl.program_id(1)
    nk = pl.num_programs(1)
    c2 = jax.lax.dot_general(
        u_ref[...], wt2_ref[...], (((1,), (0,)), ((), ())),
        preferred_element_type=jnp.float32)
    a = a_ref[...]
    bmax = jnp.max(c2, axis=1, keepdims=True)

    @pl.when(t == 0)
    def _():
        bmin_d = a - bmax
        dist = a - c2
        barg = _block_argmin(dist, bmin_d, k)

        @pl.when(k == 0)
        def _():
            arg_ref[...] = barg

        @pl.when(k != 0)
        def _():
            run_d = a - min_ref[...]
            arg_ref[...] = jnp.where(bmin_d < run_d, barg, arg_ref[...])

    @pl.when(k == 0)
    def _():
        min_ref[...] = bmax

    @pl.when(k != 0)
    def _():
        min_ref[...] = jnp.maximum(min_ref[...], bmax)

    @pl.when(k == nk - 1)
    def _():
        min_ref[...] = a - min_ref[...]


def _exact_body(u_ref, wt2_ref, a_ref, b_ref, min_ref, arg_ref):
    t = pl.program_id(0)
    k = pl.program_id(1)
    c2 = jax.lax.dot_general(
        u_ref[...], wt2_ref[...], (((1,), (0,)), ((), ())),
        preferred_element_type=jnp.float32)
    dist = (a_ref[...] + b_ref[...]) - c2
    bmin = jnp.min(dist, axis=1, keepdims=True)

    @pl.when(t == 0)
    def _():
        barg = _block_argmin(dist, bmin, k)

        @pl.when(k == 0)
        def _():
            arg_ref[...] = barg

        @pl.when(k != 0)
        def _():
            arg_ref[...] = jnp.where(bmin < min_ref[...], barg, arg_ref[...])

    @pl.when(k == 0)
    def _():
        min_ref[...] = bmin

    @pl.when(k != 0)
    def _():
        min_ref[...] = jnp.minimum(min_ref[...], bmin)


def _sort_body(key_ref, out_ref):
    keys = key_ref[...]
    n = keys.shape[1]
    pos = jax.lax.broadcasted_iota(jnp.int32, keys.shape, 1)
    i = pos

    def partner(x, j, lower):
        fwd = jnp.roll(x, -j, axis=1)
        bwd = jnp.roll(x, j, axis=1)
        return jnp.where(lower, fwd, bwd)

    k = 2
    while k <= n:
        j = k // 2
        while j >= 1:
            lower = (i & j) == 0
            pk = partner(keys, j, lower)
            pp = partner(pos, j, lower)
            asc = (i & k) == 0
            lt = (pk < keys) | ((pk == keys) & (pp < pos))
            take = lt == (lower == asc)
            keys = jnp.where(take, pk, keys)
            pos = jnp.where(take, pp, pos)
            j //= 2
        k *= 2
    out_ref[...] = pos


def _permute_body(ord_ref, w0_ref, o_ref):
    oh = (jax.lax.broadcasted_iota(jnp.int32,
                                   (ord_ref.shape[0], TOKENS_PER_ROW), 1)
          == ord_ref[...]).astype(jnp.float32)
    o_ref[...] = jnp.dot(oh, w0_ref[...], preferred_element_type=jnp.float32)


def _sc_gather(weight, idx2d, n_rows):
    mesh = plsc.VectorSubcoreMesh(core_axis_name="core",
                                  subcore_axis_name="subcore")
    window = max(128, min(256, n_rows // 32))
    wpad = jnp.pad(weight, ((0, 0), (0, 128 - DIM)))

    @pl.kernel(out_type=jax.ShapeDtypeStruct((n_rows, 128), weight.dtype),
               mesh=mesh)
    def kern(x_hbm, i_hbm, o_hbm):
        def body(i_vmem, o_vmem):
            pltpu.sync_copy(x_hbm.at[i_vmem.at[0]], o_vmem)

        pltpu.emit_pipeline(
            body,
            grid=(n_rows // window,),
            in_specs=[pl.BlockSpec((1, window), lambda i: (0, i))],
            out_specs=[pl.BlockSpec((window, 128), lambda i: (i, 0))],
            core_axis_name=("core", "subcore"),
            dimension_semantics=(pltpu.PARALLEL,),
        )(i_hbm, o_hbm)

    return kern(wpad, idx2d)[:, :DIM]


def kernel(inputs, weight):
    input_shape = inputs.shape
    flat = inputs.reshape(-1, DIM)
    wt2 = 2.0 * weight.T
    a = jnp.sum(flat ** 2, axis=1, keepdims=True)
    b = jnp.sum(weight ** 2, axis=1)[None, :]
    all_fast = (jnp.min(a) >= 16.0) & (jnp.max(b) < 2.0 ** -20)

    grid = (N_TOKENS // TBLK, NUM_K // KBLK)
    out_specs = [
        pl.BlockSpec((TBLK, 1), lambda t, k: (t, 0)),
        pl.BlockSpec((TBLK, 1), lambda t, k: (0, 0)),
    ]
    out_shape = [
        jax.ShapeDtypeStruct((N_TOKENS, 1), jnp.float32),
        jax.ShapeDtypeStruct((TBLK, 1), jnp.int32),
    ]
    cparams = pltpu.CompilerParams(
        dimension_semantics=("parallel", "arbitrary"))

    def _run_fast(ops):
        flat, wt2, a, _ = ops
        return pl.pallas_call(
            _fast_body,
            grid=grid,
            in_specs=[
                pl.BlockSpec((TBLK, DIM), lambda t, k: (t, 0)),
                pl.BlockSpec((DIM, KBLK), lambda t, k: (0, k)),
                pl.BlockSpec((TBLK, 1), lambda t, k: (t, 0)),
            ],
            out_specs=out_specs,
            out_shape=out_shape,
            compiler_params=cparams,
        )(flat, wt2, a)

    def _run_exact(ops):
        flat, wt2, a, b = ops
        return pl.pallas_call(
            _exact_body,
            grid=grid,
            in_specs=[
                pl.BlockSpec((TBLK, DIM), lambda t, k: (t, 0)),
                pl.BlockSpec((DIM, KBLK), lambda t, k: (0, k)),
                pl.BlockSpec((TBLK, 1), lambda t, k: (t, 0)),
                pl.BlockSpec((1, KBLK), lambda t, k: (0, k)),
            ],
            out_specs=out_specs,
            out_shape=out_shape,
            compiler_params=cparams,
        )(flat, wt2, a, b)

    mins, args = jax.lax.cond(all_fast, _run_fast, _run_exact,
                              (flat, wt2, a, b))

    keys = mins[:, 0].reshape(BATCH, TOKENS_PER_ROW)
    enc0 = args[:TOKENS_PER_ROW, 0]

    w0 = _sc_gather(weight, enc0.reshape(1, TOKENS_PER_ROW), TOKENS_PER_ROW)

    order = pl.pallas_call(
        _sort_body,
        in_specs=[
            pl.BlockSpec((BATCH, TOKENS_PER_ROW), lambda: (0, 0)),
        ],
        out_specs=pl.BlockSpec((BATCH, TOKENS_PER_ROW), lambda: (0, 0)),
        out_shape=jax.ShapeDtypeStruct((BATCH, TOKENS_PER_ROW), jnp.int32),
    )(keys)

    pblk = 2048
    gathered = pl.pallas_call(
        _permute_body,
        grid=(N_TOKENS // pblk,),
        in_specs=[
            pl.BlockSpec((pblk, 1), lambda i: (i, 0)),
            pl.BlockSpec((TOKENS_PER_ROW, DIM), lambda i: (0, 0)),
        ],
        out_specs=pl.BlockSpec((pblk, DIM), lambda i: (i, 0)),
        out_shape=jax.ShapeDtypeStruct((N_TOKENS, DIM), jnp.float32),
        compiler_params=pltpu.CompilerParams(
            dimension_semantics=("parallel",)),
    )(order.reshape(N_TOKENS, 1), w0)
    return gathered.reshape(input_shape)


# speedup vs baseline: 1.0277x; 1.0277x over previous
import jax
import jax.numpy as jnp
from jax.experimental import pallas as pl
from jax.experimental.pallas import tpu as pltpu
from jax.experimental.pallas import tpu_sc as plsc

NUM_K = 8192
DIM = 64
BATCH = 8
TOKENS_PER_ROW = 1024
N_TOKENS = BATCH * TOKENS_PER_ROW
KBLK = 2048
NK = NUM_K // KBLK
TBLK0 = TOKENS_PER_ROW
N1 = N_TOKENS - TBLK0
TBLK1 = N1 // 2


def _dot2(u_ref, wt2_ref):
    return jax.lax.dot_general(
        u_ref[...], wt2_ref[...], (((1,), (0,)), ((), ())),
        preferred_element_type=jnp.float32)


def _block_argmin(dist, bmin, k):
    iota = jax.lax.broadcasted_iota(
        jnp.int32, dist.shape, 1).astype(jnp.float32)
    barg_f = jnp.min(jnp.where(dist == bmin, iota, jnp.float32(2.0 ** 30)),
                     axis=1, keepdims=True)
    return barg_f.astype(jnp.int32) + k * KBLK


def _a0_fast_body(u_ref, wt2_ref, a_ref, min_ref, arg_ref):
    k = pl.program_id(0)
    nk = pl.num_programs(0)
    c2 = _dot2(u_ref, wt2_ref)
    a = a_ref[...]
    bmax = jnp.max(c2, axis=1, keepdims=True)
    bmin_d = a - bmax
    dist = a - c2
    barg = _block_argmin(dist, bmin_d, k)

    @pl.when(k == 0)
    def _():
        arg_ref[...] = barg
        min_ref[...] = bmax

    @pl.when(k != 0)
    def _():
        run_d = a - min_ref[...]
        arg_ref[...] = jnp.where(bmin_d < run_d, barg, arg_ref[...])
        min_ref[...] = jnp.maximum(min_ref[...], bmax)

    @pl.when(k == nk - 1)
    def _():
        min_ref[...] = a - min_ref[...]


def _a0_exact_body(u_ref, wt2_ref, a_ref, b_ref, min_ref, arg_ref):
    k = pl.program_id(0)
    c2 = _dot2(u_ref, wt2_ref)
    dist = (a_ref[...] + b_ref[...]) - c2
    bmin = jnp.min(dist, axis=1, keepdims=True)
    barg = _block_argmin(dist, bmin, k)

    @pl.when(k == 0)
    def _():
        arg_ref[...] = barg
        min_ref[...] = bmin

    @pl.when(k != 0)
    def _():
        arg_ref[...] = jnp.where(bmin < min_ref[...], barg, arg_ref[...])
        min_ref[...] = jnp.minimum(min_ref[...], bmin)


def _a1_fast_body(u_ref, wt2_ref, a_ref, min_ref):
    k = pl.program_id(1)
    nk = pl.num_programs(1)
    c2 = _dot2(u_ref, wt2_ref)
    bmax = jnp.max(c2, axis=1, keepdims=True)

    @pl.when(k == 0)
    def _():
        min_ref[...] = bmax

    @pl.when(k != 0)
    def _():
        min_ref[...] = jnp.maximum(min_ref[...], bmax)

    @pl.when(k == nk - 1)
    def _():
        min_ref[...] = a_ref[...] - min_ref[...]


def _a1_exact_body(u_ref, wt2_ref, a_ref, b_ref, min_ref):
    k = pl.program_id(1)
    c2 = _dot2(u_ref, wt2_ref)
    dist = (a_ref[...] + b_ref[...]) - c2
    bmin = jnp.min(dist, axis=1, keepdims=True)

    @pl.when(k == 0)
    def _():
        min_ref[...] = bmin

    @pl.when(k != 0)
    def _():
        min_ref[...] = jnp.minimum(min_ref[...], bmin)


def _sort_body(key_ref, out_ref):
    keys = key_ref[...]
    n = keys.shape[1]
    pos = jax.lax.broadcasted_iota(jnp.int32, keys.shape, 1)
    i = pos

    def partner(x, j, lower):
        fwd = jnp.roll(x, -j, axis=1)
        bwd = jnp.roll(x, j, axis=1)
        return jnp.where(lower, fwd, bwd)

    k = 2
    while k <= n:
        j = k // 2
        while j >= 1:
            lower = (i & j) == 0
            pk = partner(keys, j, lower)
            pp = partner(pos, j, lower)
            asc = (i & k) == 0
            lt = (pk < keys) | ((pk == keys) & (pp < pos))
            take = lt == (lower == asc)
            keys = jnp.where(take, pk, keys)
            pos = jnp.where(take, pp, pos)
            j //= 2
        k *= 2
    out_ref[...] = pos


def _permute_body(ord_ref, w0_ref, o_ref):
    oh = (jax.lax.broadcasted_iota(jnp.int32,
                                   (ord_ref.shape[0], TOKENS_PER_ROW), 1)
          == ord_ref[...]).astype(jnp.float32)
    o_ref[...] = jnp.dot(oh, w0_ref[...], preferred_element_type=jnp.float32)


def _sc_gather(weight, idx2d, n_rows):
    mesh = plsc.VectorSubcoreMesh(core_axis_name="core",
                                  subcore_axis_name="subcore")
    window = max(128, min(256, n_rows // 32))
    wpad = jnp.pad(weight, ((0, 0), (0, 128 - DIM)))

    @pl.kernel(out_type=jax.ShapeDtypeStruct((n_rows, 128), weight.dtype),
               mesh=mesh)
    def kern(x_hbm, i_hbm, o_hbm):
        def body(i_vmem, o_vmem):
            pltpu.sync_copy(x_hbm.at[i_vmem.at[0]], o_vmem)

        pltpu.emit_pipeline(
            body,
            grid=(n_rows // window,),
            in_specs=[pl.BlockSpec((1, window), lambda i: (0, i))],
            out_specs=[pl.BlockSpec((window, 128), lambda i: (i, 0))],
            core_axis_name=("core", "subcore"),
            dimension_semantics=(pltpu.PARALLEL,),
        )(i_hbm, o_hbm)

    return kern(wpad, idx2d)[:, :DIM]


def kernel(inputs, weight):
    input_shape = inputs.shape
    flat = inputs.reshape(-1, DIM)
    wt2 = 2.0 * weight.T
    a = jnp.sum(flat ** 2, axis=1, keepdims=True)
    b = jnp.sum(weight ** 2, axis=1)[None, :]
    all_fast = (jnp.min(a) >= 16.0) & (jnp.max(b) < 2.0 ** -20)

    flat0, flat1 = flat[:TBLK0], flat[TBLK0:]
    a0, a1 = a[:TBLK0], a[TBLK0:]

    a0_specs = dict(
        grid=(NK,),
        out_specs=[
            pl.BlockSpec((TBLK0, 1), lambda k: (0, 0)),
            pl.BlockSpec((TBLK0, 1), lambda k: (0, 0)),
        ],
        out_shape=[
            jax.ShapeDtypeStruct((TBLK0, 1), jnp.float32),
            jax.ShapeDtypeStruct((TBLK0, 1), jnp.int32),
        ],
    )
    a1_specs = dict(
        grid=(N1 // TBLK1, NK),
        out_specs=pl.BlockSpec((TBLK1, 1), lambda t, k: (t, 0)),
        out_shape=jax.ShapeDtypeStruct((N1, 1), jnp.float32),
        compiler_params=pltpu.CompilerParams(
            dimension_semantics=("parallel", "arbitrary")),
    )

    def _run_fast(ops):
        flat0, flat1, wt2, a0, a1, _ = ops
        mins0, args = pl.pallas_call(
            _a0_fast_body,
            in_specs=[
                pl.BlockSpec((TBLK0, DIM), lambda k: (0, 0)),
                pl.BlockSpec((DIM, KBLK), lambda k: (0, k)),
                pl.BlockSpec((TBLK0, 1), lambda k: (0, 0)),
            ],
            **a0_specs,
        )(flat0, wt2, a0)
        mins1 = pl.pallas_call(
            _a1_fast_body,
            in_specs=[
                pl.BlockSpec((TBLK1, DIM), lambda t, k: (t, 0)),
                pl.BlockSpec((DIM, KBLK), lambda t, k: (0, k)),
                pl.BlockSpec((TBLK1, 1), lambda t, k: (t, 0)),
            ],
            **a1_specs,
        )(flat1, wt2, a1)
        return mins0, args, mins1

    def _run_exact(ops):
        flat0, flat1, wt2, a0, a1, b = ops
        mins0, args = pl.pallas_call(
            _a0_exact_body,
            in_specs=[
                pl.BlockSpec((TBLK0, DIM), lambda k: (0, 0)),
                pl.BlockSpec((DIM, KBLK), lambda k: (0, k)),
                pl.BlockSpec((TBLK0, 1), lambda k: (0, 0)),
                pl.BlockSpec((1, KBLK), lambda k: (0, k)),
            ],
            **a0_specs,
        )(flat0, wt2, a0, b)
        mins1 = pl.pallas_call(
            _a1_exact_body,
            in_specs=[
                pl.BlockSpec((TBLK1, DIM), lambda t, k: (t, 0)),
                pl.BlockSpec((DIM, KBLK), lambda t, k: (0, k)),
                pl.BlockSpec((TBLK1, 1), lambda t, k: (t, 0)),
                pl.BlockSpec((1, KBLK), lambda t, k: (0, k)),
            ],
            **a1_specs,
        )(flat1, wt2, a1, b)
        return mins0, args, mins1

    mins0, args, mins1 = jax.lax.cond(all_fast, _run_fast, _run_exact,
                                      (flat0, flat1, wt2, a0, a1, b))

    enc0 = args[:, 0]
    w0 = _sc_gather(weight, enc0.reshape(1, TOKENS_PER_ROW), TOKENS_PER_ROW)

    keys = jnp.concatenate([mins0, mins1], axis=0)[:, 0].reshape(
        BATCH, TOKENS_PER_ROW)
    order = pl.pallas_call(
        _sort_body,
        in_specs=[
            pl.BlockSpec((BATCH, TOKENS_PER_ROW), lambda: (0, 0)),
        ],
        out_specs=pl.BlockSpec((BATCH, TOKENS_PER_ROW), lambda: (0, 0)),
        out_shape=jax.ShapeDtypeStruct((BATCH, TOKENS_PER_ROW), jnp.int32),
    )(keys)

    pblk = 2048
    gathered = pl.pallas_call(
        _permute_body,
        grid=(N_TOKENS // pblk,),
        in_specs=[
            pl.BlockSpec((pblk, 1), lambda i: (i, 0)),
            pl.BlockSpec((TOKENS_PER_ROW, DIM), lambda i: (0, 0)),
        ],
        out_specs=pl.BlockSpec((pblk, DIM), lambda i: (i, 0)),
        out_shape=jax.ShapeDtypeStruct((N_TOKENS, DIM), jnp.float32),
        compiler_params=pltpu.CompilerParams(
            dimension_semantics=("parallel",)),
    )(order.reshape(N_TOKENS, 1), w0)
    return gathered.reshape(input_shape)
